# Initial kernel scaffold; baseline (speedup 1.0000x reference)
import jax
import jax.numpy as jnp
from jax.experimental import pallas as pl


def kernel(x, w1, b1, w2, b2, w3, b3, w4, b4):
    raise NotImplementedError("write your pallas kernel here")


# speedup vs baseline: 1.0324x; 1.0324x over previous
import functools

import jax
import jax.numpy as jnp
from jax.experimental import pallas as pl
from jax.experimental.pallas import tpu as pltpu

_PACK = 4
_TB = 2048


def _mlp_body(x_ref, w1_ref, w2_ref, w3_ref, w4_ref,
              b1_ref, b2_ref, b3_ref, b4_ref, out_ref):
    h = jnp.dot(x_ref[...], w1_ref[...], preferred_element_type=jnp.float32)
    h = jnp.maximum(h + b1_ref[...], 0.0).astype(jnp.bfloat16)
    h = jnp.dot(h, w2_ref[...], preferred_element_type=jnp.float32)
    h = jnp.maximum(h + b2_ref[...], 0.0).astype(jnp.bfloat16)
    h = jnp.dot(h, w3_ref[...], preferred_element_type=jnp.float32)
    h = jnp.maximum(h + b3_ref[...], 0.0).astype(jnp.bfloat16)
    h = jnp.dot(h, w4_ref[...], preferred_element_type=jnp.float32)
    out_ref[...] = h + b4_ref[...]


def _pack_block_diag(w1, b1, w2, b2, w3, b3, w4, b4):
    w1bd = jnp.zeros((16, 64), jnp.float32)
    w2bd = jnp.zeros((64, 128), jnp.float32)
    w3bd = jnp.zeros((128, 128), jnp.float32)
    w4bd = jnp.zeros((128, 8), jnp.float32)
    for i in range(_PACK):
        w1bd = w1bd.at[4 * i:4 * i + 4, 16 * i:16 * i + 16].set(w1)
        w2bd = w2bd.at[16 * i:16 * i + 16, 32 * i:32 * i + 32].set(w2)
        w3bd = w3bd.at[32 * i:32 * i + 32, 32 * i:32 * i + 32].set(w3)
        w4bd = w4bd.at[32 * i:32 * i + 32, 2 * i:2 * i + 2].set(w4)
    b1t = jnp.tile(b1, _PACK)[None, :]
    b2t = jnp.tile(b2, _PACK)[None, :]
    b3t = jnp.tile(b3, _PACK)[None, :]
    b4t = jnp.tile(b4, _PACK)[None, :]
    return (w1bd,
            w2bd.astype(jnp.bfloat16),
            w3bd.astype(jnp.bfloat16),
            w4bd.astype(jnp.bfloat16),
            b1t, b2t, b3t, b4t)


@jax.jit
def _dqn_packed(x, w1, b1, w2, b2, w3, b3, w4, b4):
    batch = x.shape[0]
    packed = batch // _PACK
    xp = x.reshape(packed, _PACK * x.shape[1])

    w1bd, w2bd, w3bd, w4bd, b1t, b2t, b3t, b4t = _pack_block_diag(
        w1, b1, w2, b2, w3, b3, w4, b4)

    tb = min(_TB, packed)
    grid = (pl.cdiv(packed, tb),)

    resident = lambda shape: pl.BlockSpec(shape, lambda i: (0,) * len(shape))
    out = pl.pallas_call(
        _mlp_body,
        out_shape=jax.ShapeDtypeStruct((packed, 2 * _PACK), jnp.float32),
        grid=grid,
        in_specs=[
            pl.BlockSpec((tb, _PACK * x.shape[1]), lambda i: (i, 0)),
            resident(w1bd.shape),
            resident(w2bd.shape),
            resident(w3bd.shape),
            resident(w4bd.shape),
            resident(b1t.shape),
            resident(b2t.shape),
            resident(b3t.shape),
            resident(b4t.shape),
        ],
        out_specs=pl.BlockSpec((tb, 2 * _PACK), lambda i: (i, 0)),
        compiler_params=pltpu.CompilerParams(
            dimension_semantics=("parallel",)),
    )(xp, w1bd, w2bd, w3bd, w4bd, b1t, b2t, b3t, b4t)

    return out.reshape(batch, 2)


def kernel(x, w1, b1, w2, b2, w3, b3, w4, b4):
    return _dqn_packed(x, w1, b1, w2, b2, w3, b3, w4, b4)


# speedup vs baseline: 2.3779x; 2.3032x over previous
import jax
import jax.numpy as jnp
from jax.experimental import pallas as pl
from jax.experimental.pallas import tpu as pltpu

_PACK = 4
_TB = 2048


def _mlp_body(x_ref, w1_ref, w2_ref, w3_ref, w4_ref,
              b1_ref, b2_ref, b3_ref, b4_ref, out_ref):
    tb = x_ref.shape[0] // _PACK
    xp = jnp.concatenate(
        [x_ref[pl.ds(j * tb, tb), :] for j in range(_PACK)], axis=1)

    h = jnp.dot(xp, w1_ref[...], preferred_element_type=jnp.float32)
    h = jnp.maximum(h + b1_ref[...], 0.0).astype(jnp.bfloat16)
    h = jnp.dot(h, w2_ref[...], preferred_element_type=jnp.float32)
    h = jnp.maximum(h + b2_ref[...], 0.0).astype(jnp.bfloat16)
    h = jnp.dot(h, w3_ref[...], preferred_element_type=jnp.float32)
    h = jnp.maximum(h + b3_ref[...], 0.0).astype(jnp.bfloat16)

    for j in range(_PACK):
        q = jnp.dot(h[:, 32 * j:32 * j + 32], w4_ref[...],
                    preferred_element_type=jnp.float32)
        out_ref[pl.ds(j * tb, tb), :] = q + b4_ref[...]


def _pack_block_diag(w1, w2, w3, b1, b2, b3):
    w1bd = jnp.zeros((16, 64), jnp.float32)
    w2bd = jnp.zeros((64, 128), jnp.float32)
    w3bd = jnp.zeros((128, 128), jnp.float32)
    for i in range(_PACK):
        w1bd = w1bd.at[4 * i:4 * i + 4, 16 * i:16 * i + 16].set(w1)
        w2bd = w2bd.at[16 * i:16 * i + 16, 32 * i:32 * i + 32].set(w2)
        w3bd = w3bd.at[32 * i:32 * i + 32, 32 * i:32 * i + 32].set(w3)
    b1t = jnp.tile(b1, _PACK)[None, :]
    b2t = jnp.tile(b2, _PACK)[None, :]
    b3t = jnp.tile(b3, _PACK)[None, :]
    return (w1bd, w2bd.astype(jnp.bfloat16), w3bd.astype(jnp.bfloat16),
            b1t, b2t, b3t)


@jax.jit
def _dqn_fused(x, w1, b1, w2, b2, w3, b3, w4, b4):
    batch = x.shape[0]

    w1bd, w2bd, w3bd, b1t, b2t, b3t = _pack_block_diag(w1, w2, w3, b1, b2, b3)
    w4b = w4.astype(jnp.bfloat16)
    b4t = b4[None, :]

    rows = _PACK * _TB
    grid = (pl.cdiv(batch, rows),)

    resident = lambda shape: pl.BlockSpec(shape, lambda i: (0,) * len(shape))
    return pl.pallas_call(
        _mlp_body,
        out_shape=jax.ShapeDtypeStruct((batch, 2), jnp.float32),
        grid=grid,
        in_specs=[
            pl.BlockSpec((rows, x.shape[1]), lambda i: (i, 0)),
            resident(w1bd.shape),
            resident(w2bd.shape),
            resident(w3bd.shape),
            resident(w4b.shape),
            resident(b1t.shape),
            resident(b2t.shape),
            resident(b3t.shape),
            resident(b4t.shape),
        ],
        out_specs=pl.BlockSpec((rows, 2), lambda i: (i, 0)),
        compiler_params=pltpu.CompilerParams(
            dimension_semantics=("parallel",)),
    )(x, w1bd, w2bd, w3bd, w4b, b1t, b2t, b3t, b4t)


def kernel(x, w1, b1, w2, b2, w3, b3, w4, b4):
    return _dqn_fused(x, w1, b1, w2, b2, w3, b3, w4, b4)


# speedup vs baseline: 2.4775x; 1.0419x over previous
import jax
import jax.numpy as jnp
from jax.experimental import pallas as pl
from jax.experimental.pallas import tpu as pltpu

_PACK = 4
_TB = 4096


def _mlp_body(x_ref, w1_ref, w2_ref, w3_ref, w4_ref,
              b1_ref, b2_ref, b3_ref, b4_ref, out_ref):
    tb = x_ref.shape[0] // _PACK
    xp = jnp.concatenate(
        [x_ref[pl.ds(j * tb, tb), :] for j in range(_PACK)], axis=1)

    h = jnp.dot(xp, w1_ref[...], preferred_element_type=jnp.float32)
    h = jnp.maximum(h + b1_ref[...], 0.0).astype(jnp.bfloat16)
    h = jnp.dot(h, w2_ref[...], preferred_element_type=jnp.float32)
    h = jnp.maximum(h + b2_ref[...], 0.0).astype(jnp.bfloat16)
    h = jnp.dot(h, w3_ref[...], preferred_element_type=jnp.float32)
    h = jnp.maximum(h + b3_ref[...], 0.0).astype(jnp.bfloat16)

    for j in range(_PACK):
        q = jnp.dot(h[:, 32 * j:32 * j + 32], w4_ref[...],
                    preferred_element_type=jnp.float32)
        out_ref[pl.ds(j * tb, tb), :] = q + b4_ref[...]


def _pack_block_diag(w1, w2, w3, b1, b2, b3):
    w1bd = jnp.zeros((16, 64), jnp.float32)
    w2bd = jnp.zeros((64, 128), jnp.float32)
    w3bd = jnp.zeros((128, 128), jnp.float32)
    for i in range(_PACK):
        w1bd = w1bd.at[4 * i:4 * i + 4, 16 * i:16 * i + 16].set(w1)
        w2bd = w2bd.at[16 * i:16 * i + 16, 32 * i:32 * i + 32].set(w2)
        w3bd = w3bd.at[32 * i:32 * i + 32, 32 * i:32 * i + 32].set(w3)
    b1t = jnp.tile(b1, _PACK)[None, :]
    b2t = jnp.tile(b2, _PACK)[None, :]
    b3t = jnp.tile(b3, _PACK)[None, :]
    return (w1bd, w2bd.astype(jnp.bfloat16), w3bd.astype(jnp.bfloat16),
            b1t, b2t, b3t)


@jax.jit
def _dqn_fused(x, w1, b1, w2, b2, w3, b3, w4, b4):
    batch = x.shape[0]

    w1bd, w2bd, w3bd, b1t, b2t, b3t = _pack_block_diag(w1, w2, w3, b1, b2, b3)
    w4b = w4.astype(jnp.bfloat16)
    b4t = b4[None, :]

    rows = _PACK * _TB
    grid = (pl.cdiv(batch, rows),)

    resident = lambda shape: pl.BlockSpec(shape, lambda i: (0,) * len(shape))
    return pl.pallas_call(
        _mlp_body,
        out_shape=jax.ShapeDtypeStruct((batch, 2), jnp.float32),
        grid=grid,
        in_specs=[
            pl.BlockSpec((rows, x.shape[1]), lambda i: (i, 0)),
            resident(w1bd.shape),
            resident(w2bd.shape),
            resident(w3bd.shape),
            resident(w4b.shape),
            resident(b1t.shape),
            resident(b2t.shape),
            resident(b3t.shape),
            resident(b4t.shape),
        ],
        out_specs=pl.BlockSpec((rows, 2), lambda i: (i, 0)),
        compiler_params=pltpu.CompilerParams(
            dimension_semantics=("parallel",)),
    )(x, w1bd, w2bd, w3bd, w4b, b1t, b2t, b3t, b4t)


def kernel(x, w1, b1, w2, b2, w3, b3, w4, b4):
    return _dqn_fused(x, w1, b1, w2, b2, w3, b3, w4, b4)


# speedup vs baseline: 2.4779x; 1.0001x over previous
import jax
import jax.numpy as jnp
from jax.experimental import pallas as pl
from jax.experimental.pallas import tpu as pltpu

_PACK = 4
_TB = 4096


def _mlp_body(x0_ref, x1_ref, x2_ref, x3_ref,
              w1_ref, w2_ref, w3_ref, w4_ref,
              b1_ref, b2_ref, b3_ref, b4_ref, out_ref):
    tb = x0_ref.shape[0]
    xp = jnp.concatenate(
        [x0_ref[...], x1_ref[...], x2_ref[...], x3_ref[...]], axis=1)

    h = jnp.dot(xp, w1_ref[...], preferred_element_type=jnp.float32)
    h = jnp.maximum(h + b1_ref[...], 0.0).astype(jnp.bfloat16)
    h = jnp.dot(h, w2_ref[...], preferred_element_type=jnp.float32)
    h = jnp.maximum(h + b2_ref[...], 0.0).astype(jnp.bfloat16)
    h = jnp.dot(h, w3_ref[...], preferred_element_type=jnp.float32)
    h = jnp.maximum(h + b3_ref[...], 0.0).astype(jnp.bfloat16)

    for j in range(_PACK):
        q = jnp.dot(h[:, 32 * j:32 * j + 32], w4_ref[...],
                    preferred_element_type=jnp.float32)
        out_ref[pl.ds(j * tb, tb), :] = q + b4_ref[...]


def _pack_block_diag(w1, w2, w3, b1, b2, b3):
    w1bd = jnp.zeros((16, 64), jnp.float32)
    w2bd = jnp.zeros((64, 128), jnp.float32)
    w3bd = jnp.zeros((128, 128), jnp.float32)
    for i in range(_PACK):
        w1bd = w1bd.at[4 * i:4 * i + 4, 16 * i:16 * i + 16].set(w1)
        w2bd = w2bd.at[16 * i:16 * i + 16, 32 * i:32 * i + 32].set(w2)
        w3bd = w3bd.at[32 * i:32 * i + 32, 32 * i:32 * i + 32].set(w3)
    b1t = jnp.tile(b1, _PACK)[None, :]
    b2t = jnp.tile(b2, _PACK)[None, :]
    b3t = jnp.tile(b3, _PACK)[None, :]
    return (w1bd, w2bd.astype(jnp.bfloat16), w3bd.astype(jnp.bfloat16),
            b1t, b2t, b3t)


@jax.jit
def _dqn_fused(x, w1, b1, w2, b2, w3, b3, w4, b4):
    batch = x.shape[0]

    w1bd, w2bd, w3bd, b1t, b2t, b3t = _pack_block_diag(w1, w2, w3, b1, b2, b3)
    w4b = w4.astype(jnp.bfloat16)
    b4t = b4[None, :]

    rows = _PACK * _TB
    grid = (pl.cdiv(batch, rows),)

    def chunk_spec(j):
        return pl.BlockSpec((_TB, x.shape[1]),
                            lambda i, j=j: (_PACK * i + j, 0))

    resident = lambda shape: pl.BlockSpec(shape, lambda i: (0,) * len(shape))
    return pl.pallas_call(
        _mlp_body,
        out_shape=jax.ShapeDtypeStruct((batch, 2), jnp.float32),
        grid=grid,
        in_specs=[
            chunk_spec(0), chunk_spec(1), chunk_spec(2), chunk_spec(3),
            resident(w1bd.shape),
            resident(w2bd.shape),
            resident(w3bd.shape),
            resident(w4b.shape),
            resident(b1t.shape),
            resident(b2t.shape),
            resident(b3t.shape),
            resident(b4t.shape),
        ],
        out_specs=pl.BlockSpec((rows, 2), lambda i: (i, 0)),
        compiler_params=pltpu.CompilerParams(
            dimension_semantics=("parallel",)),
    )(x, x, x, x, w1bd, w2bd, w3bd, w4b, b1t, b2t, b3t, b4t)


def kernel(x, w1, b1, w2, b2, w3, b3, w4, b4):
    return _dqn_fused(x, w1, b1, w2, b2, w3, b3, w4, b4)
